# baseline (speedup 1.0000x reference)
import jax
import jax.numpy as jnp
from jax.experimental import pallas as pl
from jax.experimental.pallas import tpu as pltpu

_B = 1024
_L = 200
_N = 1000
_BLOCKS = (8, 24, 64, 136, 256, 512)
_OFFS = (0, 8, 32, 96, 232, 488)
_NB = len(_BLOCKS)


def _infectivity_body(ti_ref, tjs_ref, hist_ref, emb_ref, out_hbm,
                      gt, hist_f, out_v, out_sems):
    gt[...] = jnp.exp(tjs_ref[...] - ti_ref[...])
    hist_f[...] = hist_ref[...].astype(jnp.float32)

    def out_cp(i):
        blk = pl.ds(_OFFS[i], _BLOCKS[i])
        return pltpu.make_async_copy(out_v.at[blk, :], out_hbm.at[blk, :],
                                     out_sems.at[i])

    for i in range(_NB):
        blk = pl.ds(_OFFS[i], _BLOCKS[i])
        a = jax.lax.dot_general(
            emb_ref[blk, :], hist_f[...], (((1,), (1,)), ((), ())),
            preferred_element_type=jnp.float32)
        out_v[blk, :] = jax.lax.dot_general(
            a, gt[...], (((1,), (1,)), ((), ())),
            preferred_element_type=jnp.float32)
        out_cp(i).start()

    for i in range(_NB):
        out_cp(i).wait()


def kernel(ti, tjs, ci, cjs, emb_weight):
    del ci
    hist = cjs.reshape(_L, _N)
    out2d = pl.pallas_call(
        _infectivity_body,
        in_specs=[
            pl.BlockSpec(memory_space=pltpu.MemorySpace.VMEM),
            pl.BlockSpec(memory_space=pltpu.MemorySpace.VMEM),
            pl.BlockSpec(memory_space=pltpu.MemorySpace.VMEM),
            pl.BlockSpec(memory_space=pltpu.MemorySpace.VMEM),
        ],
        out_specs=pl.BlockSpec(memory_space=pltpu.MemorySpace.HBM),
        out_shape=jax.ShapeDtypeStruct((_N, _B), jnp.float32),
        scratch_shapes=[
            pltpu.VMEM((_B, _L), jnp.float32),
            pltpu.VMEM((_L, _N), jnp.float32),
            pltpu.VMEM((_N, _B), jnp.float32),
            pltpu.SemaphoreType.DMA((_NB,)),
        ],
    )(ti, tjs, hist, emb_weight)
    return out2d[:, :, None]


# speedup vs baseline: 1.0714x; 1.0714x over previous
import jax
import jax.numpy as jnp
from jax.experimental import pallas as pl
from jax.experimental.pallas import tpu as pltpu

_B = 1024
_L = 200
_N = 1000
_BM = 200
_NB = _N // _BM


def _infectivity_body(ti_ref, tjs_ref, hist_ref, emb_ref, out_hbm,
                      gt, hist_f, out_v, out_sems):
    hist_f[...] = hist_ref[...].astype(jnp.float32)
    blk0 = pl.ds(0, _BM)
    a0 = jax.lax.dot_general(
        emb_ref[blk0, :], hist_f[...], (((1,), (1,)), ((), ())),
        preferred_element_type=jnp.float32)
    gt[...] = jnp.exp(tjs_ref[...] - ti_ref[...])

    def out_cp(i):
        blk = pl.ds(i * _BM, _BM)
        return pltpu.make_async_copy(out_v.at[blk, :], out_hbm.at[blk, :],
                                     out_sems.at[i])

    out_v[blk0, :] = jax.lax.dot_general(
        a0, gt[...], (((1,), (1,)), ((), ())),
        preferred_element_type=jnp.float32)
    out_cp(0).start()

    for i in range(1, _NB):
        blk = pl.ds(i * _BM, _BM)
        a = jax.lax.dot_general(
            emb_ref[blk, :], hist_f[...], (((1,), (1,)), ((), ())),
            preferred_element_type=jnp.float32)
        out_v[blk, :] = jax.lax.dot_general(
            a, gt[...], (((1,), (1,)), ((), ())),
            preferred_element_type=jnp.float32)
        out_cp(i).start()

    for i in range(_NB):
        out_cp(i).wait()


def kernel(ti, tjs, ci, cjs, emb_weight):
    del ci
    hist = cjs.reshape(_L, _N)
    out2d = pl.pallas_call(
        _infectivity_body,
        in_specs=[
            pl.BlockSpec(memory_space=pltpu.MemorySpace.VMEM),
            pl.BlockSpec(memory_space=pltpu.MemorySpace.VMEM),
            pl.BlockSpec(memory_space=pltpu.MemorySpace.VMEM),
            pl.BlockSpec(memory_space=pltpu.MemorySpace.VMEM),
        ],
        out_specs=pl.BlockSpec(memory_space=pltpu.MemorySpace.HBM),
        out_shape=jax.ShapeDtypeStruct((_N, _B), jnp.float32),
        scratch_shapes=[
            pltpu.VMEM((_B, _L), jnp.float32),
            pltpu.VMEM((_L, _N), jnp.float32),
            pltpu.VMEM((_N, _B), jnp.float32),
            pltpu.SemaphoreType.DMA((_NB,)),
        ],
    )(ti, tjs, hist, emb_weight)
    return out2d[:, :, None]
